# baseline (speedup 1.0000x reference)
import functools

import jax
import jax.numpy as jnp
from jax import lax
from jax.experimental import pallas as pl
from jax.experimental.pallas import tpu as pltpu
from jax.experimental.pallas import tpu_sc as plsc

_PT = 128
_CT = 256


def _make_sc_kernel(B, C, P):
    NC = 2
    n_pc = P // _PT
    n_cc = C // _CT
    n_cg = _CT // 16

    mesh = plsc.VectorSubcoreMesh(core_axis_name="c", subcore_axis_name="s")

    @functools.partial(
        pl.kernel,
        mesh=mesh,
        out_type=jax.ShapeDtypeStruct((B, P, C), jnp.float32),
        compiler_params=pltpu.CompilerParams(needs_layout_passes=False),
        scratch_types=[
            pltpu.VMEM((_CT, _PT + 1), jnp.float32),
            pltpu.VMEM((_PT, _CT), jnp.float32),
            pltpu.VMEM((_PT, _CT), jnp.float32),
        ],
    )
    def k(x_hbm, pe_hbm, o_hbm, xbuf, pebuf, outbuf):
        b = lax.axis_index("s") * NC + lax.axis_index("c")
        iota16 = jnp.arange(16, dtype=jnp.int32)

        def do_chunk(ci, carry):
            p0 = pl.multiple_of((ci // n_cc) * _PT, _PT)
            c0 = pl.multiple_of((ci % n_cc) * _CT, _CT)
            pltpu.sync_copy(x_hbm.at[b, pl.ds(c0, _CT), pl.ds(p0, _PT)], xbuf.at[:, pl.ds(0, _PT)])
            pltpu.sync_copy(pe_hbm.at[pl.ds(p0, _PT), pl.ds(c0, _CT)], pebuf)

            def do_p(p, carry2):
                idx_p = jnp.full((16,), p, dtype=jnp.int32)
                for cg in range(n_cg):
                    cg0 = cg * 16
                    v = plsc.load_gather(xbuf, [cg0 + iota16, idx_p])
                    outbuf[p, pl.ds(cg0, 16)] = v + pebuf[p, pl.ds(cg0, 16)]
                return carry2

            lax.fori_loop(0, _PT, do_p, carry)
            pltpu.sync_copy(outbuf, o_hbm.at[b, pl.ds(p0, _PT), pl.ds(c0, _CT)])
            return carry

        lax.fori_loop(0, n_pc * n_cc, do_chunk, 0)

    return k


def kernel(image_feature, pe_table):
    B, C, H, W = image_feature.shape
    P = H * W
    x = image_feature.reshape(B, C, P)
    return _make_sc_kernel(B, C, P)(x, pe_table)


# speedup vs baseline: 5.9356x; 5.9356x over previous
import jax
import jax.numpy as jnp
from jax.experimental import pallas as pl
from jax.experimental.pallas import tpu as pltpu

_CH = 256
_NBUF = 8


def _make_body(B, C, P):
    J = P // _CH
    N = B * J

    def body(x_ref, pe_ref, o_ref, inbuf, outbuf, insem, outsem):
        def in_copy(m, slot):
            b = m // J
            j = m % J
            return pltpu.make_async_copy(
                x_ref.at[b, :, pl.ds(j * _CH, _CH)],
                inbuf.at[slot],
                insem.at[slot],
            )

        def out_copy(m, slot):
            b = m // J
            j = m % J
            return pltpu.make_async_copy(
                outbuf.at[slot],
                o_ref.at[b, pl.ds(j * _CH, _CH), :],
                outsem.at[slot],
            )

        n = pl.program_id(0)
        slot = jax.lax.rem(n, _NBUF)

        @pl.when(n == 0)
        def _():
            for k in range(_NBUF - 1):
                in_copy(k, k).start()

        @pl.when(n >= _NBUF)
        def _():
            out_copy(n - _NBUF, slot).wait()

        in_copy(n, slot).wait()
        j = jax.lax.rem(n, J)
        outbuf[slot] = inbuf[slot].T + pe_ref[pl.ds(j * _CH, _CH), :]
        out_copy(n, slot).start()

        @pl.when(n + _NBUF - 1 < N)
        def _():
            in_copy(n + _NBUF - 1, jax.lax.rem(n + _NBUF - 1, _NBUF)).start()

        @pl.when(n == N - 1)
        def _():
            for m in range(N - _NBUF, N):
                out_copy(m, m % _NBUF).wait()

    return body, N


def kernel(image_feature, pe_table):
    B, C, H, W = image_feature.shape
    P = H * W
    x = image_feature.reshape(B, C, P)
    body, N = _make_body(B, C, P)
    return pl.pallas_call(
        body,
        grid=(N,),
        in_specs=[
            pl.BlockSpec(memory_space=pltpu.MemorySpace.HBM),
            pl.BlockSpec(memory_space=pltpu.MemorySpace.VMEM),
        ],
        out_specs=pl.BlockSpec(memory_space=pltpu.MemorySpace.HBM),
        out_shape=jax.ShapeDtypeStruct((B, P, C), image_feature.dtype),
        scratch_shapes=[
            pltpu.VMEM((_NBUF, C, _CH), jnp.float32),
            pltpu.VMEM((_NBUF, _CH, C), jnp.float32),
            pltpu.SemaphoreType.DMA((_NBUF,)),
            pltpu.SemaphoreType.DMA((_NBUF,)),
        ],
    )(x, pe_table)
